# baseline (speedup 1.0000x reference)
import functools

import jax
import jax.numpy as jnp
from jax import lax
from jax.experimental import pallas as pl
from jax.experimental.pallas import tpu as pltpu
from jax.experimental.pallas import tpu_sc as plsc

B = 16384
D = 128
V = 1000
VPAD = 1024

SC_ROWS = 8192
TC_ROWS = B - SC_ROWS

NW = 32
BPW = SC_ROWS // NW
CHUNK = 128
NCHUNK = BPW // CHUNK
L = 16
NLOADERS = 8

TC_BLOCK = 1024

_mesh = plsc.VectorSubcoreMesh(core_axis_name="c", subcore_axis_name="s")


@functools.partial(
    pl.kernel,
    mesh=_mesh,
    out_type=jax.ShapeDtypeStruct((SC_ROWS,), jnp.float32),
    scratch_types=[
        pltpu.VMEM((NCHUNK, CHUNK), jnp.int32),
        pltpu.VMEM((2, CHUNK, D), jnp.float32),
        pltpu.VMEM((2, CHUNK, D), jnp.float32),
        pltpu.VMEM((BPW,), jnp.float32),
        pltpu.VMEM((L * L,), jnp.float32),
        pltpu.VMEM_SHARED((V, D), jnp.float32),
        pltpu.SemaphoreType.DMA,
        pltpu.SemaphoreType.DMA,
        pltpu.SemaphoreType.DMA,
        pltpu.SemaphoreType.DMA,
        pltpu.SemaphoreType.DMA,
        pltpu.SemaphoreType.DMA,
    ],
    compiler_params=pltpu.CompilerParams(
        needs_layout_passes=False, skip_device_barrier=True),
)
def _sc_reward_dot(psi_hbm, ids_hbm, w_hbm, out_hbm,
                   idx_v, w_v, psi_v, out_v, t_v, tbl_s,
                   sem_t, sem_i, sem_p0, sem_p1, sem_w0, sem_w1):
    sid = lax.axis_index("s")
    wid = sid * 2 + lax.axis_index("c")
    base = wid * BPW
    sems_p = (sem_p0, sem_p1)
    sems_w = (sem_w0, sem_w1)

    id_cps = []
    for c in range(NCHUNK):
        cp = pltpu.make_async_copy(
            ids_hbm.at[pl.ds(base + c * CHUNK, CHUNK)], idx_v.at[c], sem_i)
        cp.start()
        id_cps.append(cp)

    for t in range(NLOADERS):
        r0 = t * 128
        nr = min(128, V - r0)

        @pl.when(sid == t)
        def _load_table(r0=r0, nr=nr):
            pltpu.make_async_copy(
                w_hbm.at[pl.ds(r0, nr)], tbl_s.at[pl.ds(r0, nr)], sem_t).start()

    def start_psi(c):
        b = c & 1
        p = pltpu.make_async_copy(
            psi_hbm.at[pl.ds(base + c * CHUNK, CHUNK)], psi_v.at[b], sems_p[b])
        p.start()
        return p

    psi_cps = [start_psi(0), start_psi(1)]
    for cp in id_cps:
        cp.wait()
    for t in range(NLOADERS):
        r0 = t * 128
        nr = min(128, V - r0)

        @pl.when(sid == t)
        def _wait_table(r0=r0, nr=nr):
            pltpu.make_async_copy(
                w_hbm.at[pl.ds(r0, nr)], tbl_s.at[pl.ds(r0, nr)], sem_t).wait()

    plsc.subcore_barrier()

    def start_w(c):
        b = c & 1
        w = pltpu.make_async_copy(tbl_s.at[idx_v.at[c]], w_v.at[b], sems_w[b])
        w.start()
        return w

    cps = [(psi_cps[0], start_w(0)), (psi_cps[1], start_w(1))]

    for c in range(NCHUNK):
        b = c & 1
        p_cp, w_cp = cps[b]
        p_cp.wait()
        w_cp.wait()

        def group_body(g, carry, b=b, c=c):
            def row_body(r, carry2, b=b, g=g):
                row = g * L + r
                acc = psi_v[b, row, pl.ds(0, L)] * w_v[b, row, pl.ds(0, L)]
                for k in range(1, D // L):
                    acc = acc + psi_v[b, row, pl.ds(k * L, L)] * w_v[b, row, pl.ds(k * L, L)]
                plsc.store_scatter(t_v, [lax.iota(jnp.int32, L) * L + r], acc)
                return carry2

            lax.fori_loop(0, L, row_body, 0, unroll=2)
            v = t_v[pl.ds(0, L)]
            for l in range(1, L):
                v = v + t_v[pl.ds(l * L, L)]
            out_v[pl.ds(c * CHUNK + g * L, L)] = v
            return carry

        lax.fori_loop(0, CHUNK // L, group_body, 0)
        if c + 2 < NCHUNK:
            cps[b] = (start_psi(c + 2), start_w(c + 2))

    pltpu.sync_copy(out_v, out_hbm.at[pl.ds(base, BPW)])


def _tc_body(psi_ref, ids_ref, w_ref, out_ref):
    ids_row = ids_ref[...].reshape(1, TC_BLOCK)
    q_t = lax.dot_general(
        w_ref[...], psi_ref[...].astype(jnp.bfloat16),
        (((1,), (1,)), ((), ())),
        preferred_element_type=jnp.float32)
    iota0 = lax.broadcasted_iota(jnp.int32, (VPAD, TC_BLOCK), 0)
    masked = jnp.where(iota0 == ids_row, q_t, 0.0)
    out_ref[...] = jnp.sum(masked, axis=0)


_tc_gather_dot = pl.pallas_call(
    _tc_body,
    grid=(TC_ROWS // TC_BLOCK,),
    in_specs=[
        pl.BlockSpec((TC_BLOCK, D), lambda i: (i, 0)),
        pl.BlockSpec((TC_BLOCK,), lambda i: (i,)),
        pl.BlockSpec((VPAD, D), lambda i: (0, 0)),
    ],
    out_specs=pl.BlockSpec((TC_BLOCK,), lambda i: (i,)),
    out_shape=jax.ShapeDtypeStruct((TC_ROWS,), jnp.float32),
)


def kernel(psi, task_ids, weights):
    ids = task_ids.astype(jnp.int32)
    sc_out = _sc_reward_dot(psi, ids, weights)
    w_pad = jnp.zeros((VPAD, D), jnp.bfloat16).at[:V].set(
        weights.astype(jnp.bfloat16))
    tc_out = _tc_gather_dot(
        lax.slice_in_dim(psi, SC_ROWS, B), lax.slice_in_dim(ids, SC_ROWS, B),
        w_pad)
    return jnp.concatenate([sc_out, tc_out])


# speedup vs baseline: 1.1535x; 1.1535x over previous
import functools

import jax
import jax.numpy as jnp
from jax import lax
from jax.experimental import pallas as pl
from jax.experimental.pallas import tpu as pltpu
from jax.experimental.pallas import tpu_sc as plsc

B = 16384
D = 128
V = 1000
VPAD = 1024

SC_ROWS = 8192
TC_ROWS = B - SC_ROWS

NW = 32
BPW = SC_ROWS // NW
CHUNK = 128
NCHUNK = BPW // CHUNK
L = 16
NLOADERS = 8

TC_BLOCK = 1024

_mesh = plsc.VectorSubcoreMesh(core_axis_name="c", subcore_axis_name="s")


@functools.partial(
    pl.kernel,
    mesh=_mesh,
    out_type=jax.ShapeDtypeStruct((SC_ROWS,), jnp.float32),
    scratch_types=[
        pltpu.VMEM((NCHUNK, CHUNK), jnp.int32),
        pltpu.VMEM((2, CHUNK, D), jnp.float32),
        pltpu.VMEM((2, CHUNK, D), jnp.float32),
        pltpu.VMEM((BPW,), jnp.float32),
        pltpu.VMEM((L * L,), jnp.float32),
        pltpu.VMEM_SHARED((V, D), jnp.float32),
        pltpu.SemaphoreType.DMA,
        pltpu.SemaphoreType.DMA,
        pltpu.SemaphoreType.DMA,
        pltpu.SemaphoreType.DMA,
        pltpu.SemaphoreType.DMA,
        pltpu.SemaphoreType.DMA,
    ],
    compiler_params=pltpu.CompilerParams(
        needs_layout_passes=False, skip_device_barrier=True),
)
def _sc_reward_dot(psi_hbm, ids_hbm, w_hbm, out_hbm,
                   idx_v, w_v, psi_v, out_v, t_v, tbl_s,
                   sem_t, sem_i, sem_p0, sem_p1, sem_w0, sem_w1):
    sid = lax.axis_index("s")
    wid = sid * 2 + lax.axis_index("c")
    base = wid * BPW
    sems_p = (sem_p0, sem_p1)
    sems_w = (sem_w0, sem_w1)

    id_cps = []
    for c in range(NCHUNK):
        cp = pltpu.make_async_copy(
            ids_hbm.at[pl.ds(base + c * CHUNK, CHUNK)], idx_v.at[c], sem_i)
        cp.start()
        id_cps.append(cp)

    for t in range(NLOADERS):
        r0 = t * 128
        nr = min(128, V - r0)

        @pl.when(sid == t)
        def _load_table(r0=r0, nr=nr):
            pltpu.make_async_copy(
                w_hbm.at[pl.ds(r0, nr)], tbl_s.at[pl.ds(r0, nr)], sem_t).start()

    def start_psi(c):
        b = c & 1
        p = pltpu.make_async_copy(
            psi_hbm.at[pl.ds(base + c * CHUNK, CHUNK)], psi_v.at[b], sems_p[b])
        p.start()
        return p

    psi_cps = [start_psi(0), start_psi(1)]
    for cp in id_cps:
        cp.wait()
    for t in range(NLOADERS):
        r0 = t * 128
        nr = min(128, V - r0)

        @pl.when(sid == t)
        def _wait_table(r0=r0, nr=nr):
            pltpu.make_async_copy(
                w_hbm.at[pl.ds(r0, nr)], tbl_s.at[pl.ds(r0, nr)], sem_t).wait()

    plsc.subcore_barrier()

    def start_w(c):
        b = c & 1
        w = pltpu.make_async_copy(tbl_s.at[idx_v.at[c]], w_v.at[b], sems_w[b])
        w.start()
        return w

    cps = [(psi_cps[0], start_w(0)), (psi_cps[1], start_w(1))]

    for c in range(NCHUNK):
        b = c & 1
        p_cp, w_cp = cps[b]
        p_cp.wait()
        w_cp.wait()

        def group_body(g, carry, b=b, c=c):
            def row_body(r, carry2, b=b, g=g):
                row = g * L + r
                acc = psi_v[b, row, pl.ds(0, L)] * w_v[b, row, pl.ds(0, L)]
                for k in range(1, D // L):
                    acc = acc + psi_v[b, row, pl.ds(k * L, L)] * w_v[b, row, pl.ds(k * L, L)]
                plsc.store_scatter(t_v, [lax.iota(jnp.int32, L) * L + r], acc)
                return carry2

            lax.fori_loop(0, L, row_body, 0, unroll=2)
            v = t_v[pl.ds(0, L)]
            for l in range(1, L):
                v = v + t_v[pl.ds(l * L, L)]
            out_v[pl.ds(c * CHUNK + g * L, L)] = v
            return carry

        lax.fori_loop(0, CHUNK // L, group_body, 0)
        if c + 2 < NCHUNK:
            cps[b] = (start_psi(c + 2), start_w(c + 2))

    pltpu.sync_copy(out_v, out_hbm.at[pl.ds(base, BPW)])


def _tc_body(psi_ref, ids_ref, w_ref, out_ref):
    ids_row = ids_ref[...].reshape(1, TC_BLOCK)
    q_t = lax.dot_general(
        w_ref[...], psi_ref[...].astype(jnp.bfloat16),
        (((1,), (1,)), ((), ())),
        preferred_element_type=jnp.float32)
    iota0 = lax.broadcasted_iota(jnp.int32, (VPAD, TC_BLOCK), 0)
    masked = jnp.where(iota0 == ids_row, q_t, 0.0)
    out_ref[...] = jnp.sum(masked, axis=0)


_tc_gather_dot = pl.pallas_call(
    _tc_body,
    grid=(TC_ROWS // TC_BLOCK,),
    in_specs=[
        pl.BlockSpec((TC_BLOCK, D), lambda i: (SC_ROWS // TC_BLOCK + i, 0)),
        pl.BlockSpec((TC_BLOCK,), lambda i: (SC_ROWS // TC_BLOCK + i,)),
        pl.BlockSpec((VPAD, D), lambda i: (0, 0)),
    ],
    out_specs=pl.BlockSpec((TC_BLOCK,), lambda i: (i,)),
    out_shape=jax.ShapeDtypeStruct((TC_ROWS,), jnp.float32),
)


def kernel(psi, task_ids, weights):
    ids = task_ids.astype(jnp.int32)
    sc_out = _sc_reward_dot(psi, ids, weights)
    w_pad = jnp.zeros((VPAD, D), jnp.bfloat16).at[:V].set(
        weights.astype(jnp.bfloat16))
    tc_out = _tc_gather_dot(psi, ids, w_pad)
    return jnp.concatenate([sc_out, tc_out])
